# baseline (speedup 1.0000x reference)
import functools

import jax
import jax.numpy as jnp
from jax.experimental import pallas as pl
from jax.experimental.pallas import tpu as pltpu

S = 16
C = 768
HW = 576
BR = 288
NB = HW // BR


def _body(thr_ref, w_ref, xs_ref, xo_ref, o_ref):
    thr = thr_ref[0]
    m0 = (jnp.abs(w_ref[0:1, :]) < thr)[:, None, :]
    m1 = (jnp.abs(w_ref[1:2, :]) < thr)[:, None, :]
    xs = xs_ref[...]
    xo = xo_ref[...]
    o_ref[0] = jnp.where(m0, xo, xs)
    o_ref[1] = jnp.where(m1, xs, xo)


@jax.jit
def _exchange(xt, w, thr):
    return pl.pallas_call(
        _body,
        grid=(8, NB),
        in_specs=[
            pl.BlockSpec(memory_space=pltpu.SMEM),
            pl.BlockSpec((2, C), lambda s, i: (0, 0)),
            pl.BlockSpec((1, BR, C), lambda s, i: (s, i, 0)),
            pl.BlockSpec((1, BR, C), lambda s, i: (s + 8, i, 0)),
        ],
        out_specs=pl.BlockSpec((2, 1, BR, C), lambda s, i: (0, s, i, 0)),
        out_shape=jax.ShapeDtypeStruct((2, 8, HW, C), jnp.float32),
        compiler_params=pltpu.CompilerParams(
            dimension_semantics=("parallel", "parallel"),
        ),
    )(thr, w, xt, xt)


def kernel(x, bn_weight, bn_threshold):
    xt = x.transpose(0, 2, 3, 1).reshape(S, HW, C)
    thr = jnp.full((1,), bn_threshold, dtype=jnp.float32)
    out = _exchange(xt, bn_weight, thr)
    return out.reshape(S, 24, 24, C).transpose(0, 3, 1, 2)


# speedup vs baseline: 1.1763x; 1.1763x over previous
import jax
import jax.numpy as jnp
from jax.experimental import pallas as pl
from jax.experimental.pallas import tpu as pltpu

S = 16
C = 768
HW = 576
CH = 288
NCH = HW // CH
NU = 8 * NCH
NBUF = 4


def _body(thr_ref, w_ref, x_hbm, o_hbm, ibuf, obuf, rsem, wsem):
    thr = thr_ref[0]
    m0 = jnp.abs(w_ref[0:1, :]) < thr
    m1 = jnp.abs(w_ref[1:2, :]) < thr

    def start_read(u):
        s, i = u // NCH, u % NCH
        sl = u % NBUF
        pltpu.make_async_copy(
            x_hbm.at[s, pl.ds(i * CH, CH)], ibuf.at[sl, 0], rsem.at[sl, 0]
        ).start()
        pltpu.make_async_copy(
            x_hbm.at[s + 8, pl.ds(i * CH, CH)], ibuf.at[sl, 1], rsem.at[sl, 1]
        ).start()

    def wait_read(u):
        s, i = u // NCH, u % NCH
        sl = u % NBUF
        pltpu.make_async_copy(
            x_hbm.at[s, pl.ds(i * CH, CH)], ibuf.at[sl, 0], rsem.at[sl, 0]
        ).wait()
        pltpu.make_async_copy(
            x_hbm.at[s + 8, pl.ds(i * CH, CH)], ibuf.at[sl, 1], rsem.at[sl, 1]
        ).wait()

    def start_write(u):
        s, i = u // NCH, u % NCH
        sl = u % NBUF
        pltpu.make_async_copy(
            obuf.at[sl, 0], o_hbm.at[0, s, pl.ds(i * CH, CH)], wsem.at[sl, 0]
        ).start()
        pltpu.make_async_copy(
            obuf.at[sl, 1], o_hbm.at[1, s, pl.ds(i * CH, CH)], wsem.at[sl, 1]
        ).start()

    def wait_write(u):
        s, i = u // NCH, u % NCH
        sl = u % NBUF
        pltpu.make_async_copy(
            obuf.at[sl, 0], o_hbm.at[0, s, pl.ds(i * CH, CH)], wsem.at[sl, 0]
        ).wait()
        pltpu.make_async_copy(
            obuf.at[sl, 1], o_hbm.at[1, s, pl.ds(i * CH, CH)], wsem.at[sl, 1]
        ).wait()

    for v in range(min(NBUF, NU)):
        start_read(v)
    for u in range(NU):
        wait_read(u)
        if u >= NBUF:
            wait_write(u - NBUF)
        sl = u % NBUF
        xs = ibuf[sl, 0]
        xo = ibuf[sl, 1]
        obuf[sl, 0] = jnp.where(m0, xo, xs)
        obuf[sl, 1] = jnp.where(m1, xs, xo)
        start_write(u)
        if u + NBUF < NU:
            start_read(u + NBUF)
    for u in range(max(NU - NBUF, 0), NU):
        wait_write(u)


@jax.jit
def _exchange(xt, w, thr):
    return pl.pallas_call(
        _body,
        in_specs=[
            pl.BlockSpec(memory_space=pltpu.SMEM),
            pl.BlockSpec(memory_space=pltpu.VMEM),
            pl.BlockSpec(memory_space=pl.ANY),
        ],
        out_specs=pl.BlockSpec(memory_space=pl.ANY),
        out_shape=jax.ShapeDtypeStruct((2, 8, HW, C), jnp.float32),
        scratch_shapes=[
            pltpu.VMEM((NBUF, 2, CH, C), jnp.float32),
            pltpu.VMEM((NBUF, 2, CH, C), jnp.float32),
            pltpu.SemaphoreType.DMA((NBUF, 2)),
            pltpu.SemaphoreType.DMA((NBUF, 2)),
        ],
    )(thr, w, xt)


def kernel(x, bn_weight, bn_threshold):
    xt = x.transpose(0, 2, 3, 1).reshape(S, HW, C)
    thr = jnp.full((1,), bn_threshold, dtype=jnp.float32)
    out = _exchange(xt, bn_weight, thr)
    return out.reshape(S, 24, 24, C).transpose(0, 3, 1, 2)


# speedup vs baseline: 1.1796x; 1.0028x over previous
import jax
import jax.numpy as jnp
from jax.experimental import pallas as pl
from jax.experimental.pallas import tpu as pltpu

S = 16
C = 768
HW = 576
CH = 144
NCH = HW // CH
NU = 8 * NCH
NBUF = 8


def _body(thr_ref, w_ref, x_hbm, o_hbm, ibuf, obuf, rsem, wsem):
    thr = thr_ref[0]
    m0 = jnp.abs(w_ref[0:1, :]) < thr
    m1 = jnp.abs(w_ref[1:2, :]) < thr

    def start_read(u):
        s, i = u // NCH, u % NCH
        sl = u % NBUF
        pltpu.make_async_copy(
            x_hbm.at[s, pl.ds(i * CH, CH)], ibuf.at[sl, 0], rsem.at[sl, 0]
        ).start()
        pltpu.make_async_copy(
            x_hbm.at[s + 8, pl.ds(i * CH, CH)], ibuf.at[sl, 1], rsem.at[sl, 1]
        ).start()

    def wait_read(u):
        s, i = u // NCH, u % NCH
        sl = u % NBUF
        pltpu.make_async_copy(
            x_hbm.at[s, pl.ds(i * CH, CH)], ibuf.at[sl, 0], rsem.at[sl, 0]
        ).wait()
        pltpu.make_async_copy(
            x_hbm.at[s + 8, pl.ds(i * CH, CH)], ibuf.at[sl, 1], rsem.at[sl, 1]
        ).wait()

    def start_write(u):
        s, i = u // NCH, u % NCH
        sl = u % NBUF
        pltpu.make_async_copy(
            obuf.at[sl, 0], o_hbm.at[0, s, pl.ds(i * CH, CH)], wsem.at[sl, 0]
        ).start()
        pltpu.make_async_copy(
            obuf.at[sl, 1], o_hbm.at[1, s, pl.ds(i * CH, CH)], wsem.at[sl, 1]
        ).start()

    def wait_write(u):
        s, i = u // NCH, u % NCH
        sl = u % NBUF
        pltpu.make_async_copy(
            obuf.at[sl, 0], o_hbm.at[0, s, pl.ds(i * CH, CH)], wsem.at[sl, 0]
        ).wait()
        pltpu.make_async_copy(
            obuf.at[sl, 1], o_hbm.at[1, s, pl.ds(i * CH, CH)], wsem.at[sl, 1]
        ).wait()

    for v in range(min(NBUF, NU)):
        start_read(v)
    for u in range(NU):
        wait_read(u)
        if u >= NBUF:
            wait_write(u - NBUF)
        sl = u % NBUF
        xs = ibuf[sl, 0]
        xo = ibuf[sl, 1]
        obuf[sl, 0] = jnp.where(m0, xo, xs)
        obuf[sl, 1] = jnp.where(m1, xs, xo)
        start_write(u)
        if u + NBUF < NU:
            start_read(u + NBUF)
    for u in range(max(NU - NBUF, 0), NU):
        wait_write(u)


@jax.jit
def _exchange(xt, w, thr):
    return pl.pallas_call(
        _body,
        in_specs=[
            pl.BlockSpec(memory_space=pltpu.SMEM),
            pl.BlockSpec(memory_space=pltpu.VMEM),
            pl.BlockSpec(memory_space=pl.ANY),
        ],
        out_specs=pl.BlockSpec(memory_space=pl.ANY),
        out_shape=jax.ShapeDtypeStruct((2, 8, HW, C), jnp.float32),
        scratch_shapes=[
            pltpu.VMEM((NBUF, 2, CH, C), jnp.float32),
            pltpu.VMEM((NBUF, 2, CH, C), jnp.float32),
            pltpu.SemaphoreType.DMA((NBUF, 2)),
            pltpu.SemaphoreType.DMA((NBUF, 2)),
        ],
    )(thr, w, xt)


def kernel(x, bn_weight, bn_threshold):
    xt = x.transpose(0, 2, 3, 1).reshape(S, HW, C)
    thr = jnp.full((1,), bn_threshold, dtype=jnp.float32)
    out = _exchange(xt, bn_weight, thr)
    return out.reshape(S, 24, 24, C).transpose(0, 3, 1, 2)
